# baseline (speedup 1.0000x reference)
import functools

import jax
import jax.numpy as jnp
from jax import lax
from jax.experimental import pallas as pl
from jax.experimental.pallas import tpu as pltpu
from jax.experimental.pallas import tpu_sc as plsc

B = 4096
D = 32
L = 16
NC = 2
NS = 16
NW = NC * NS
BPW = B // NW

_mesh = plsc.VectorSubcoreMesh(core_axis_name="c", subcore_axis_name="s")


@functools.partial(
    pl.kernel,
    mesh=_mesh,
    out_type=jax.ShapeDtypeStruct((B,), jnp.float32),
    scratch_types=[
        pltpu.VMEM((BPW,), jnp.int32),
        pltpu.VMEM((BPW,), jnp.int32),
        pltpu.VMEM((BPW, D), jnp.float32),
        pltpu.VMEM((BPW, D), jnp.float32),
        pltpu.VMEM((BPW,), jnp.float32),
        pltpu.SemaphoreType.DMA,
        pltpu.SemaphoreType.DMA,
    ],
    compiler_params=pltpu.CompilerParams(
        needs_layout_passes=False, use_tc_tiling_on_sc=False
    ),
)
def _neg_sampling_dot(idx_c_hbm, idx_w_hbm, ctx_hbm, word_hbm, out_hbm,
                      idx_c_v, idx_w_v, rows_c, rows_w, acc, sem_c, sem_w):
    wid = lax.axis_index("s") * NC + lax.axis_index("c")
    base = wid * BPW

    pltpu.sync_copy(idx_c_hbm.at[pl.ds(base, BPW)], idx_c_v)
    pltpu.sync_copy(idx_w_hbm.at[pl.ds(base, BPW)], idx_w_v)

    cp_c = pltpu.async_copy(ctx_hbm.at[idx_c_v], rows_c, sem_c)
    cp_w = pltpu.async_copy(word_hbm.at[idx_w_v], rows_w, sem_w)
    cp_c.wait()
    cp_w.wait()

    lane = lax.iota(jnp.int32, L)
    for blk in range(BPW // L):
        rows16 = lane + blk * L
        acc_v = jnp.zeros((L,), jnp.float32)
        for d in range(D):
            cols = jnp.full((L,), d, jnp.int32)
            wv = plsc.load_gather(rows_w, [rows16, cols])
            cv = plsc.load_gather(rows_c, [rows16, cols])
            acc_v = acc_v + wv * cv
        acc[pl.ds(blk * L, L)] = acc_v

    pltpu.sync_copy(acc, out_hbm.at[pl.ds(base, BPW)])


def kernel(inputs, word_embeddings, context_embeddings):
    idx_c = inputs[:, 0].astype(jnp.int32)
    idx_w = inputs[:, 1].astype(jnp.int32)
    return _neg_sampling_dot(idx_c, idx_w, context_embeddings, word_embeddings)
